# baseline (speedup 1.0000x reference)
import functools

import jax
import jax.numpy as jnp
from jax import lax
from jax.experimental import pallas as pl
from jax.experimental.pallas import tpu as pltpu
from jax.experimental.pallas import tpu_sc as plsc

_B, _D, _H, _DO, _K = 1024, 256, 65536, 256, 128

_BB = 256
_HB = 2048

_CH = 128
_NCK = _H // _CH

_NC, _NS, _L = 2, 16, 16
_NW = _NC * _NS
_ROWS_W = _B // _NW
_MCAP = 4096

_NB = 64
_NBS = 26


def _scores_body(x_ref, w_ref, o_ref, m_ref):
    s = lax.dot_general(
        x_ref[...], w_ref[...], (((1,), (1,)), ((), ())),
        preferred_element_type=jnp.float32)
    nck_b = _HB // _CH
    s3 = s.reshape(_BB, nck_b, _CH)
    o_ref[...] = jnp.transpose(s3, (1, 0, 2))
    m_ref[...] = jnp.max(s3, axis=2).reshape(1, _BB, nck_b)


def _compute_scores(x, w_in, bh):
    return pl.pallas_call(
        _scores_body,
        grid=(bh // _BB, _H // _HB),
        in_specs=[
            pl.BlockSpec((_BB, _D), lambda i, j: (i, 0)),
            pl.BlockSpec((_HB, _D), lambda i, j: (j, 0)),
        ],
        out_specs=[
            pl.BlockSpec((_HB // _CH, _BB, _CH), lambda i, j: (j, i, 0)),
            pl.BlockSpec((1, _BB, _HB // _CH), lambda i, j: (j, i, 0)),
        ],
        out_shape=[
            jax.ShapeDtypeStruct((_NCK, bh, _CH), jnp.float32),
            jax.ShapeDtypeStruct((_H // _HB, bh, _HB // _CH), jnp.float32),
        ],
    )(x, w_in)


def _key_of(v):
    b = lax.bitcast_convert_type(v, jnp.int32)
    m = lax.shift_right_arithmetic(b, 31)
    return b ^ lax.shift_right_logical(m, 1)


def _val_of(k):
    m = lax.shift_right_arithmetic(k, 31)
    return lax.bitcast_convert_type(k ^ lax.shift_right_logical(m, 1),
                                    jnp.float32)


def _gelu(x):
    z = x * 0.7071067811865476
    az = jnp.abs(z)
    t = 1.0 / (1.0 + 0.3275911 * az)
    e = jnp.exp(-(az * az))
    poly = ((((1.061405429 * t - 1.453152027) * t + 1.421413741) * t
             - 0.284496736) * t + 0.254829592) * t
    erf_abs = 1.0 - poly * e
    erf = jnp.where(z < 0, -erf_abs, erf_abs)
    return 0.5 * x * (1.0 + erf)


def _bisect(memb_v, count, rank, lo0, hi0, iters, lanes):
    nv = lax.shift_right_arithmetic(count + (_L - 1), 4)

    def bis(it, lh):
        lo, hi = lh
        mid = lo + lax.shift_right_logical(hi - lo, 1)

        def cnt(i2, acc):
            kv = memb_v[pl.ds(pl.multiple_of(i2 * _L, _L), _L)]
            valid = (i2 * _L + lanes) < count
            return acc + jnp.where((kv > mid) & valid, 1, 0)
        accv = lax.fori_loop(0, nv, cnt, jnp.zeros((_L,), jnp.int32))
        geq = jnp.sum(accv) >= rank
        return (jnp.where(geq, mid + 1, lo), jnp.where(geq, hi, mid))
    t_key, _ = lax.fori_loop(0, iters, bis, (lo0, hi0))
    return t_key


def _select_rank(load_key, n_vecs, rank, hist_v, memb_v, lanes, laneoff, ones):
    zeros = jnp.zeros((_L,), jnp.int32)

    def zb(g, c):
        hist_v[pl.ds(g * _L, _L)] = zeros
        return c
    lax.fori_loop(0, _NB, zb, 0, unroll=8)

    def hb(i, c):
        bkt = lax.shift_right_arithmetic(load_key(i), _NBS) + _NB // 2
        plsc.addupdate_scatter(hist_v, [laneoff + bkt], ones)
        return c
    lax.fori_loop(0, n_vecs, hb, 0, unroll=8)

    carry = jnp.int32(0)
    e_sel = jnp.int32(-1)
    cg_sel = jnp.int32(-1)
    for g in reversed(range(_NB // _L)):
        tot = hist_v[pl.ds(g * _L, _L)]
        for l in range(1, 16):
            tot = tot + hist_v[pl.ds(l * _NB + g * _L, _L)]
        rv = lax.rev(tot, (0,))
        incl = jnp.cumsum(rv)
        above = lax.rev(incl, (0,)) - tot + carry
        m = (above < rank) & (above + tot >= rank)
        e_sel = jnp.maximum(e_sel, jnp.max(jnp.where(m, g * _L + lanes, -1)))
        cg_sel = jnp.maximum(cg_sel, jnp.max(jnp.where(m, above, -1)))
        carry = carry + jnp.sum(tot)

    r_in = rank - cg_sel
    e_signed = e_sel - _NB // 2

    def cb(i, off):
        k = load_key(i)
        msk = lax.shift_right_arithmetic(k, _NBS) == e_signed
        plsc.store_compressed(memb_v.at[pl.ds(off, _L)], k, mask=msk)
        return off + jnp.sum(jnp.where(msk, 1, 0))
    count = lax.fori_loop(0, n_vecs, cb, jnp.int32(0))

    lo0 = lax.shift_left(e_signed, _NBS)
    hi0 = lo0 | jnp.int32((1 << _NBS) - 1)
    return _bisect(memb_v, count, r_in, lo0, hi0, _NBS, lanes)


def _sc_body(bh, sch_hbm, mx_hbm, w_hbm, out_hbm,
             mxs_v, idxs_v, gat_v, hist_v, membk_v, membp_v,
             sval_v, coef_v, seli_v, wo_v, oring_v,
             sem_m, sem_g, sem_w, sem_o):
    rows_w = bh // _NW
    lbh = bh.bit_length() - 1
    wid = lax.axis_index("s") * _NC + lax.axis_index("c")
    base_row = wid * rows_w
    lanes = lax.iota(jnp.int32, _L)
    laneoff = lanes * _NB
    ones = jnp.ones((_L,), jnp.int32)
    nvx = _NCK // _L

    def mxfire(i, c):
        pltpu.async_copy(
            mx_hbm.at[i, pl.ds(base_row * _L, rows_w * _L)],
            mxs_v.at[i], sem_m)
        return c
    lax.fori_loop(0, _H // _HB, mxfire, 0)

    def mxdrain(i, c):
        pltpu.make_async_copy(
            mx_hbm.at[i, pl.ds(base_row * _L, rows_w * _L)],
            mxs_v.at[i], sem_m).wait()
        return c
    lax.fori_loop(0, _H // _HB, mxdrain, 0)

    def stage_a(j, mk_acc):
        def lk(i):
            return _key_of(mxs_v[i, pl.ds(j * _L, _L)])

        m_key = _select_rank(lk, nvx, jnp.int32(_K),
                             hist_v, membk_v, lanes, laneoff, ones)

        rowb = base_row + j

        def c1(i, off):
            msk = lk(i) > m_key
            ids = (i * _L + lanes) * bh + rowb
            plsc.store_compressed(
                idxs_v.at[pl.ds(j * 256 + off, _L)], ids, mask=msk)
            return off + jnp.sum(jnp.where(msk, 1, 0))
        ngt = lax.fori_loop(0, nvx, c1, jnp.int32(0), unroll=4)

        def c2(i, off):
            msk = lk(i) == m_key
            mi = jnp.where(msk, 1, 0)
            exc = plsc.cumsum(mi) - mi
            keep = msk & ((off + exc) < _K)
            ids = (i * _L + lanes) * bh + rowb
            offc = jnp.minimum(off, _K)
            plsc.store_compressed(
                idxs_v.at[pl.ds(j * 256 + offc, _L)], ids, mask=keep)
            return off + jnp.sum(jnp.where(keep, 1, 0))
        lax.fori_loop(0, nvx, c2, ngt, unroll=4)

        mk_a, mk_b = mk_acc
        mk_a = jnp.where(lanes == j, jnp.full((_L,), m_key, jnp.int32), mk_a)
        mk_b = jnp.where(lanes == j - _L, jnp.full((_L,), m_key, jnp.int32),
                         mk_b)
        return (mk_a, mk_b)

    mk_a, mk_b = lax.fori_loop(
        0, rows_w, stage_a,
        (jnp.zeros((_L,), jnp.int32), jnp.zeros((_L,), jnp.int32)))

    def chunk_copy(j):
        return pltpu.make_async_copy(
            sch_hbm.at[idxs_v.at[pl.ds(j * 256, _K)]],
            gat_v.at[j & 1], sem_g.at[j & 1])

    pltpu.async_copy(
        sch_hbm.at[idxs_v.at[pl.ds(0, _K)]], gat_v.at[0], sem_g.at[0])

    def stage_b(j, c):
        @pl.when(j < rows_w)
        def _():
            p = j & 1
            chunk_copy(j).wait()

            @pl.when(j < rows_w - 1)
            def _():
                pltpu.async_copy(
                    sch_hbm.at[idxs_v.at[pl.ds((j + 1) * 256, _K)]],
                    gat_v.at[(j + 1) & 1], sem_g.at[(j + 1) & 1])

            m_key = jnp.max(jnp.where(lanes == j, mk_a, jnp.int32(-2**31)))
            m_key = jnp.maximum(
                m_key,
                jnp.max(jnp.where(lanes == j - _L, mk_b, jnp.int32(-2**31))))

            def cmp_b(i, off):
                row = lax.shift_right_arithmetic(i, 3)
                col = lax.shift_left(i & 7, 4)
                k = _key_of(gat_v[p, row, pl.ds(col, _L)])
                msk = k >= m_key
                offc = jnp.minimum(off, _MCAP - _L)
                plsc.store_compressed(membk_v.at[pl.ds(offc, _L)], k,
                                      mask=msk)
                plsc.store_compressed(membp_v.at[pl.ds(offc, _L)],
                                      i * _L + lanes, mask=msk)
                return off + jnp.sum(jnp.where(msk, 1, 0))
            count = lax.fori_loop(0, _K * _CH // _L, cmp_b, jnp.int32(0),
                                  unroll=8)
            count = jnp.minimum(count, _MCAP)

            t_key = _bisect(membk_v, count, jnp.int32(_K),
                            m_key, jnp.int32(2**31 - 1), 32, lanes)

            nv2 = lax.shift_right_arithmetic(count + (_L - 1), 4)

            def sel_pass(eq):
                def sp(i2, off):
                    kv = membk_v[pl.ds(pl.multiple_of(i2 * _L, _L), _L)]
                    pv = membp_v[pl.ds(pl.multiple_of(i2 * _L, _L), _L)]
                    valid = (i2 * _L + lanes) < count
                    if eq:
                        msk = (kv == t_key) & valid
                    else:
                        msk = (kv > t_key) & valid
                    mi = jnp.where(msk, 1, 0)
                    exc = plsc.cumsum(mi) - mi
                    keep = msk & ((off + exc) < _K)
                    offc = jnp.minimum(off, _K)
                    plsc.store_compressed(sval_v.at[pl.ds(offc, _L)],
                                          _val_of(kv), mask=keep)
                    pvs = lax.shift_right_logical(pv, 7) & jnp.int32(_K - 1)
                    cid = plsc.load_gather(idxs_v, [j * 256 + pvs])
                    gid = (lax.shift_right_logical(cid, lbh) * _CH
                           + (pv & (_CH - 1)))
                    plsc.store_compressed(seli_v.at[p, pl.ds(offc, _L)],
                                          gid, mask=keep)
                    return off + jnp.sum(jnp.where(keep, 1, 0))
                return sp
            ngt = lax.fori_loop(0, nv2, sel_pass(False), jnp.int32(0))
            lax.fori_loop(0, nv2, sel_pass(True), ngt)

            def gel(r, cc):
                coef_v[pl.ds(p * _K + r * _L, _L)] = _gelu(
                    sval_v[pl.ds(r * _L, _L)])
                return cc
            lax.fori_loop(0, _K // _L, gel, 0)

        @pl.when(j > 0)
        def _():
            q = (j - 1) & 1
            acc = tuple(jnp.zeros((_L,), jnp.float32) for _ in range(16))
            for h in range(2):
                pltpu.make_async_copy(
                    w_hbm.at[seli_v.at[q, pl.ds(h * 64, 64)]],
                    wo_v.at[h], sem_w.at[h]).wait()

                def wacc(rr, accs):
                    cf = plsc.load_gather(
                        coef_v,
                        [jnp.full((_L,), q * _K + h * 64, jnp.int32) + rr])
                    return tuple(
                        accs[cc] + cf * wo_v[h, rr, pl.ds(cc * _L, _L)]
                        for cc in range(16))
                acc = lax.fori_loop(0, 64, wacc, acc)

            @pl.when(j - 1 >= 2)
            def _():
                pltpu.make_async_copy(
                    oring_v.at[q], out_hbm.at[base_row + j - 3],
                    sem_o.at[q]).wait()
            for cc in range(16):
                oring_v[q, pl.ds(cc * _L, _L)] = acc[cc]
            pltpu.async_copy(
                oring_v.at[q], out_hbm.at[base_row + j - 1], sem_o.at[q])

        @pl.when(j < rows_w)
        def _():
            p = j & 1
            for h in range(2):
                pltpu.async_copy(
                    w_hbm.at[seli_v.at[p, pl.ds(h * 64, 64)]],
                    wo_v.at[h], sem_w.at[h])
        return c

    lax.fori_loop(0, rows_w + 1, stage_b, 0)

    pltpu.make_async_copy(
        oring_v.at[0], out_hbm.at[base_row + rows_w - 2], sem_o.at[0]).wait()
    pltpu.make_async_copy(
        oring_v.at[1], out_hbm.at[base_row + rows_w - 1], sem_o.at[1]).wait()


def _make_sc(bh):
    rows_w = bh // _NW
    return functools.partial(
        pl.kernel,
        out_type=jax.ShapeDtypeStruct((bh, _DO), jnp.float32),
        mesh=plsc.VectorSubcoreMesh(core_axis_name="c", subcore_axis_name="s"),
        compiler_params=pltpu.CompilerParams(needs_layout_passes=False),
        scratch_types=[
            pltpu.VMEM((_H // _HB, rows_w * _L), jnp.float32),
            pltpu.VMEM((rows_w * 256,), jnp.int32),
            pltpu.VMEM((2, _K, _CH), jnp.float32),
            pltpu.VMEM((_NB * _L,), jnp.int32),
            pltpu.VMEM((_MCAP + _L,), jnp.int32),
            pltpu.VMEM((_MCAP + _L,), jnp.int32),
            pltpu.VMEM((_K + _L,), jnp.float32),
            pltpu.VMEM((2 * _K,), jnp.float32),
            pltpu.VMEM((2, _K + _L), jnp.int32),
            pltpu.VMEM((2, 64, _DO), jnp.float32),
            pltpu.VMEM((2, _DO), jnp.float32),
            pltpu.SemaphoreType.DMA,
            pltpu.SemaphoreType.DMA((2,)),
            pltpu.SemaphoreType.DMA((2,)),
            pltpu.SemaphoreType.DMA((2,)),
        ],
    )(functools.partial(_sc_body, bh))


_NHALF = 8
_BH = _B // _NHALF
_sc_half = _make_sc(_BH)


def kernel(x_B_D, in_weight, out_weight):
    outs = []
    for h in range(_NHALF):
        xh = lax.slice_in_dim(x_B_D, h * _BH, (h + 1) * _BH, axis=0)
        scores, mx = _compute_scores(xh, in_weight, _BH)
        outs.append(_sc_half(scores.reshape(_NCK * _BH, _CH),
                             mx.reshape(_H // _HB, _BH * _L),
                             out_weight))
    return jnp.concatenate(outs, axis=0)


# speedup vs baseline: 1.9502x; 1.9502x over previous
import functools

import jax
import jax.numpy as jnp
from jax import lax
from jax.experimental import pallas as pl
from jax.experimental.pallas import tpu as pltpu
from jax.experimental.pallas import tpu_sc as plsc

_B, _D, _H, _DO, _K = 1024, 256, 65536, 256, 128

_BB = 256
_HB = 2048

_CH = 128
_NCK = _H // _CH

_NC, _NS, _L = 2, 16, 16
_NW = _NC * _NS
_ROWS_W = _B // _NW
_MCAP = 4096

_NB = 64
_NBS = 26


def _scores_body(x_ref, w_ref, o_ref, m_ref):
    s = lax.dot_general(
        x_ref[...], w_ref[...], (((1,), (1,)), ((), ())),
        preferred_element_type=jnp.float32)
    nck_b = _HB // _CH
    s3 = s.reshape(_BB, nck_b, _CH)
    o_ref[...] = jnp.transpose(s3, (1, 0, 2))
    m_ref[...] = jnp.max(s3, axis=2).reshape(1, _BB, nck_b)


def _compute_scores(x, w_in, bh):
    return pl.pallas_call(
        _scores_body,
        grid=(bh // _BB, _H // _HB),
        in_specs=[
            pl.BlockSpec((_BB, _D), lambda i, j: (i, 0)),
            pl.BlockSpec((_HB, _D), lambda i, j: (j, 0)),
        ],
        out_specs=[
            pl.BlockSpec((_HB // _CH, _BB, _CH), lambda i, j: (j, i, 0)),
            pl.BlockSpec((1, _BB, _HB // _CH), lambda i, j: (j, i, 0)),
        ],
        out_shape=[
            jax.ShapeDtypeStruct((_NCK, bh, _CH), jnp.float32),
            jax.ShapeDtypeStruct((_H // _HB, bh, _HB // _CH), jnp.float32),
        ],
    )(x, w_in)


def _key_of(v):
    b = lax.bitcast_convert_type(v, jnp.int32)
    m = lax.shift_right_arithmetic(b, 31)
    return b ^ lax.shift_right_logical(m, 1)


def _val_of(k):
    m = lax.shift_right_arithmetic(k, 31)
    return lax.bitcast_convert_type(k ^ lax.shift_right_logical(m, 1),
                                    jnp.float32)


def _gelu(x):
    z = x * 0.7071067811865476
    az = jnp.abs(z)
    t = 1.0 / (1.0 + 0.3275911 * az)
    e = jnp.exp(-(az * az))
    poly = ((((1.061405429 * t - 1.453152027) * t + 1.421413741) * t
             - 0.284496736) * t + 0.254829592) * t
    erf_abs = 1.0 - poly * e
    erf = jnp.where(z < 0, -erf_abs, erf_abs)
    return 0.5 * x * (1.0 + erf)


def _bisect(memb_v, count, rank, lo0, hi0, iters, lanes):
    nv = lax.shift_right_arithmetic(count + (_L - 1), 4)

    def bis(it, lh):
        lo, hi = lh
        mid = lo + lax.shift_right_logical(hi - lo, 1)

        def cnt(i2, acc):
            kv = memb_v[pl.ds(pl.multiple_of(i2 * _L, _L), _L)]
            valid = (i2 * _L + lanes) < count
            return acc + jnp.where((kv > mid) & valid, 1, 0)
        accv = lax.fori_loop(0, nv, cnt, jnp.zeros((_L,), jnp.int32))
        geq = jnp.sum(accv) >= rank
        return (jnp.where(geq, mid + 1, lo), jnp.where(geq, hi, mid))
    t_key, _ = lax.fori_loop(0, iters, bis, (lo0, hi0))
    return t_key


def _select_rank(load_key, n_vecs, rank, hist_v, memb_v, lanes, laneoff, ones):
    zeros = jnp.zeros((_L,), jnp.int32)

    def zb(g, c):
        hist_v[pl.ds(g * _L, _L)] = zeros
        return c
    lax.fori_loop(0, _NB, zb, 0, unroll=8)

    def hb(i, c):
        bkt = lax.shift_right_arithmetic(load_key(i), _NBS) + _NB // 2
        plsc.addupdate_scatter(hist_v, [laneoff + bkt], ones)
        return c
    lax.fori_loop(0, n_vecs, hb, 0, unroll=8)

    carry = jnp.int32(0)
    e_sel = jnp.int32(-1)
    cg_sel = jnp.int32(-1)
    for g in reversed(range(_NB // _L)):
        tot = hist_v[pl.ds(g * _L, _L)]
        for l in range(1, 16):
            tot = tot + hist_v[pl.ds(l * _NB + g * _L, _L)]
        rv = lax.rev(tot, (0,))
        incl = jnp.cumsum(rv)
        above = lax.rev(incl, (0,)) - tot + carry
        m = (above < rank) & (above + tot >= rank)
        e_sel = jnp.maximum(e_sel, jnp.max(jnp.where(m, g * _L + lanes, -1)))
        cg_sel = jnp.maximum(cg_sel, jnp.max(jnp.where(m, above, -1)))
        carry = carry + jnp.sum(tot)

    r_in = rank - cg_sel
    e_signed = e_sel - _NB // 2

    def cb(i, off):
        k = load_key(i)
        msk = lax.shift_right_arithmetic(k, _NBS) == e_signed
        plsc.store_compressed(memb_v.at[pl.ds(off, _L)], k, mask=msk)
        return off + jnp.sum(jnp.where(msk, 1, 0))
    count = lax.fori_loop(0, n_vecs, cb, jnp.int32(0))

    lo0 = lax.shift_left(e_signed, _NBS)
    hi0 = lo0 | jnp.int32((1 << _NBS) - 1)
    return _bisect(memb_v, count, r_in, lo0, hi0, _NBS, lanes)


def _sc_body(bh, sch_hbm, mx_hbm, w_hbm, out_hbm,
             mxs_v, idxs_v, gat_v, hist_v, membk_v, membp_v,
             sval_v, coef_v, seli_v, wo_v, oring_v,
             sem_m, sem_g, sem_w, sem_o):
    rows_w = bh // _NW
    lbh = bh.bit_length() - 1
    wid = lax.axis_index("s") * _NC + lax.axis_index("c")
    base_row = wid * rows_w
    lanes = lax.iota(jnp.int32, _L)
    laneoff = lanes * _NB
    ones = jnp.ones((_L,), jnp.int32)
    nvx = _NCK // _L

    def mxfire(i, c):
        pltpu.async_copy(
            mx_hbm.at[i, pl.ds(base_row * _L, rows_w * _L)],
            mxs_v.at[i], sem_m)
        return c
    lax.fori_loop(0, _H // _HB, mxfire, 0)

    def mxdrain(i, c):
        pltpu.make_async_copy(
            mx_hbm.at[i, pl.ds(base_row * _L, rows_w * _L)],
            mxs_v.at[i], sem_m).wait()
        return c
    lax.fori_loop(0, _H // _HB, mxdrain, 0)

    def stage_a(j, mk_acc):
        def lk(i):
            return _key_of(mxs_v[i, pl.ds(j * _L, _L)])

        m_key = _select_rank(lk, nvx, jnp.int32(_K),
                             hist_v, membk_v, lanes, laneoff, ones)

        rowb = base_row + j

        def c1(i, off):
            msk = lk(i) > m_key
            ids = (i * _L + lanes) * bh + rowb
            plsc.store_compressed(
                idxs_v.at[pl.ds(j * 256 + off, _L)], ids, mask=msk)
            return off + jnp.sum(jnp.where(msk, 1, 0))
        ngt = lax.fori_loop(0, nvx, c1, jnp.int32(0), unroll=4)

        def c2(i, off):
            msk = lk(i) == m_key
            mi = jnp.where(msk, 1, 0)
            exc = plsc.cumsum(mi) - mi
            keep = msk & ((off + exc) < _K)
            ids = (i * _L + lanes) * bh + rowb
            offc = jnp.minimum(off, _K)
            plsc.store_compressed(
                idxs_v.at[pl.ds(j * 256 + offc, _L)], ids, mask=keep)
            return off + jnp.sum(jnp.where(keep, 1, 0))
        lax.fori_loop(0, nvx, c2, ngt, unroll=4)

        mk_a, mk_b = mk_acc
        mk_a = jnp.where(lanes == j, jnp.full((_L,), m_key, jnp.int32), mk_a)
        mk_b = jnp.where(lanes == j - _L, jnp.full((_L,), m_key, jnp.int32),
                         mk_b)
        return (mk_a, mk_b)

    mk_a, mk_b = lax.fori_loop(
        0, rows_w, stage_a,
        (jnp.zeros((_L,), jnp.int32), jnp.zeros((_L,), jnp.int32)))

    def chunk_copy(j):
        return pltpu.make_async_copy(
            sch_hbm.at[idxs_v.at[pl.ds(j * 256, _K)]],
            gat_v.at[j & 1], sem_g.at[j & 1])

    pltpu.async_copy(
        sch_hbm.at[idxs_v.at[pl.ds(0, _K)]], gat_v.at[0], sem_g.at[0])

    def stage_b(j, c):
        @pl.when(j < rows_w)
        def _():
            p = j & 1
            chunk_copy(j).wait()

            @pl.when(j < rows_w - 1)
            def _():
                pltpu.async_copy(
                    sch_hbm.at[idxs_v.at[pl.ds((j + 1) * 256, _K)]],
                    gat_v.at[(j + 1) & 1], sem_g.at[(j + 1) & 1])

            m_key = jnp.max(jnp.where(lanes == j, mk_a, jnp.int32(-2**31)))
            m_key = jnp.maximum(
                m_key,
                jnp.max(jnp.where(lanes == j - _L, mk_b, jnp.int32(-2**31))))

            def cmp_b(i, off):
                row = lax.shift_right_arithmetic(i, 3)
                col = lax.shift_left(i & 7, 4)
                k = _key_of(gat_v[p, row, pl.ds(col, _L)])
                msk = k >= m_key
                offc = jnp.minimum(off, _MCAP - _L)
                plsc.store_compressed(membk_v.at[pl.ds(offc, _L)], k,
                                      mask=msk)
                plsc.store_compressed(membp_v.at[pl.ds(offc, _L)],
                                      i * _L + lanes, mask=msk)
                return off + jnp.sum(jnp.where(msk, 1, 0))
            count = lax.fori_loop(0, _K * _CH // _L, cmp_b, jnp.int32(0),
                                  unroll=8)
            count = jnp.minimum(count, _MCAP)

            t_key = _bisect(membk_v, count, jnp.int32(_K),
                            m_key, jnp.int32(2**31 - 1), 32, lanes)

            nv2 = lax.shift_right_arithmetic(count + (_L - 1), 4)

            def sel_pass(eq):
                def sp(i2, off):
                    kv = membk_v[pl.ds(pl.multiple_of(i2 * _L, _L), _L)]
                    pv = membp_v[pl.ds(pl.multiple_of(i2 * _L, _L), _L)]
                    valid = (i2 * _L + lanes) < count
                    if eq:
                        msk = (kv == t_key) & valid
                    else:
                        msk = (kv > t_key) & valid
                    mi = jnp.where(msk, 1, 0)
                    exc = plsc.cumsum(mi) - mi
                    keep = msk & ((off + exc) < _K)
                    offc = jnp.minimum(off, _K)
                    plsc.store_compressed(sval_v.at[pl.ds(offc, _L)],
                                          _val_of(kv), mask=keep)
                    pvs = lax.shift_right_logical(pv, 7) & jnp.int32(_K - 1)
                    cid = plsc.load_gather(idxs_v, [j * 256 + pvs])
                    gid = (lax.shift_right_logical(cid, lbh) * _CH
                           + (pv & (_CH - 1)))
                    plsc.store_compressed(seli_v.at[p, pl.ds(offc, _L)],
                                          gid, mask=keep)
                    return off + jnp.sum(jnp.where(keep, 1, 0))
                return sp
            ngt = lax.fori_loop(0, nv2, sel_pass(False), jnp.int32(0))
            lax.fori_loop(0, nv2, sel_pass(True), ngt)

            def gel(r, cc):
                coef_v[pl.ds(p * _K + r * _L, _L)] = _gelu(
                    sval_v[pl.ds(r * _L, _L)])
                return cc
            lax.fori_loop(0, _K // _L, gel, 0)

        @pl.when(j > 0)
        def _():
            q = (j - 1) & 1
            acc = tuple(jnp.zeros((_L,), jnp.float32) for _ in range(16))
            for h in range(2):
                pltpu.make_async_copy(
                    w_hbm.at[seli_v.at[q, pl.ds(h * 64, 64)]],
                    wo_v.at[h], sem_w.at[h]).wait()

                def wacc(rr, accs):
                    cf = plsc.load_gather(
                        coef_v,
                        [jnp.full((_L,), q * _K + h * 64, jnp.int32) + rr])
                    return tuple(
                        accs[cc] + cf * wo_v[h, rr, pl.ds(cc * _L, _L)]
                        for cc in range(16))
                acc = lax.fori_loop(0, 64, wacc, acc)

            @pl.when(j - 1 >= 2)
            def _():
                pltpu.make_async_copy(
                    oring_v.at[q], out_hbm.at[base_row + j - 3],
                    sem_o.at[q]).wait()
            for cc in range(16):
                oring_v[q, pl.ds(cc * _L, _L)] = acc[cc]
            pltpu.async_copy(
                oring_v.at[q], out_hbm.at[base_row + j - 1], sem_o.at[q])

        @pl.when(j < rows_w)
        def _():
            p = j & 1
            for h in range(2):
                pltpu.async_copy(
                    w_hbm.at[seli_v.at[p, pl.ds(h * 64, 64)]],
                    wo_v.at[h], sem_w.at[h])
        return c

    lax.fori_loop(0, rows_w + 1, stage_b, 0)

    pltpu.make_async_copy(
        oring_v.at[0], out_hbm.at[base_row + rows_w - 2], sem_o.at[0]).wait()
    pltpu.make_async_copy(
        oring_v.at[1], out_hbm.at[base_row + rows_w - 1], sem_o.at[1]).wait()


def _make_sc(bh):
    rows_w = bh // _NW
    return functools.partial(
        pl.kernel,
        out_type=jax.ShapeDtypeStruct((bh, _DO), jnp.float32),
        mesh=plsc.VectorSubcoreMesh(core_axis_name="c", subcore_axis_name="s"),
        compiler_params=pltpu.CompilerParams(needs_layout_passes=False),
        scratch_types=[
            pltpu.VMEM((_H // _HB, rows_w * _L), jnp.float32),
            pltpu.VMEM((rows_w * 256,), jnp.int32),
            pltpu.VMEM((2, _K, _CH), jnp.float32),
            pltpu.VMEM((_NB * _L,), jnp.int32),
            pltpu.VMEM((_MCAP + _L,), jnp.int32),
            pltpu.VMEM((_MCAP + _L,), jnp.int32),
            pltpu.VMEM((_K + _L,), jnp.float32),
            pltpu.VMEM((2 * _K,), jnp.float32),
            pltpu.VMEM((2, _K + _L), jnp.int32),
            pltpu.VMEM((2, 64, _DO), jnp.float32),
            pltpu.VMEM((2, _DO), jnp.float32),
            pltpu.SemaphoreType.DMA,
            pltpu.SemaphoreType.DMA((2,)),
            pltpu.SemaphoreType.DMA((2,)),
            pltpu.SemaphoreType.DMA((2,)),
        ],
    )(functools.partial(_sc_body, bh))


_NHALF = 4
_BH = _B // _NHALF
_sc_half = _make_sc(_BH)


def kernel(x_B_D, in_weight, out_weight):
    outs = []
    for h in range(_NHALF):
        xh = lax.slice_in_dim(x_B_D, h * _BH, (h + 1) * _BH, axis=0)
        scores, mx = _compute_scores(xh, in_weight, _BH)
        outs.append(_sc_half(scores.reshape(_NCK * _BH, _CH),
                             mx.reshape(_H // _HB, _BH * _L),
                             out_weight))
    return jnp.concatenate(outs, axis=0)
